# baseline (speedup 1.0000x reference)
import functools

import jax
import jax.numpy as jnp
from jax import lax
from jax.experimental import pallas as pl
from jax.experimental.pallas import tpu as pltpu
from jax.experimental.pallas import tpu_sc as plsc

N = 10000
E = 320000
D = 128
H = 64
C = 2
W2P = 16
WD = 0.0005

NC = 2
NS = 16
LANES = 16
NW = NC * NS
EPW = E // NW
NP = 10240
RPS = NP // NS
ZR = 40

K1 = 400
NCH1 = EPW // K1
K2 = 1000
NCH2 = EPW // K2

_mesh = plsc.VectorSubcoreMesh(core_axis_name="c", subcore_axis_name="s")


def _seg_kernel(width, kchunk, nchunk, nbuf, extract_cols):
    nsub = width // LANES
    if extract_cols:
        out_type = jax.ShapeDtypeStruct((2 * NC, NP), jnp.float32)
    else:
        out_type = jax.ShapeDtypeStruct((NC * NP, width), jnp.float32)
    scratch = [
        pltpu.VMEM_SHARED((NP, width), jnp.float32),
        pltpu.VMEM((2, RPS), jnp.float32),
        pltpu.VMEM((EPW,), jnp.int32),
        pltpu.VMEM((nchunk, kchunk), jnp.int32),
        pltpu.VMEM((EPW,), jnp.float32),
    ]
    scratch += [pltpu.VMEM((kchunk, width), jnp.float32)
                for _ in range(nbuf)]
    scratch += [pltpu.SemaphoreType.DMA] * (2 + 2 * nbuf)

    @functools.partial(
        pl.kernel,
        out_type=out_type,
        mesh=_mesh,
        compiler_params=pltpu.CompilerParams(
            needs_layout_passes=False, use_tc_tiling_on_sc=False),
        scratch_types=scratch,
    )
    def body(table, eih, ewh, zh, out, acc, lbuf, srcb, dstb, ewb, *rest):
        rows = list(rest[:nbuf])
        mi, mz = rest[nbuf], rest[nbuf + 1]
        semg = list(rest[nbuf + 2:nbuf + 2 + nbuf])
        sems = list(rest[nbuf + 2 + nbuf:nbuf + 2 + 2 * nbuf])

        c = lax.axis_index("c")
        s = lax.axis_index("s")
        base = (s * NC + c) * EPW

        stg = [pltpu.async_copy(eih.at[0, pl.ds(base, EPW)], srcb, mi),
               pltpu.async_copy(ewh.at[pl.ds(base, EPW)], ewb, mi)]
        stg += [pltpu.async_copy(eih.at[1, pl.ds(base + k * kchunk, kchunk)],
                                 dstb.at[k], mi) for k in range(nchunk)]
        zdsc = [pltpu.async_copy(zh,
                                 acc.at[pl.ds(s * RPS + j * ZR, ZR)], mz)
                for j in range(RPS // ZR)]
        for dsc in zdsc:
            dsc.wait()
        for dsc in stg:
            dsc.wait()
        plsc.subcore_barrier()

        gat_dsc = [None] * nbuf
        sct_dsc = [None] * nbuf

        def issue_gather(k):
            gat_dsc[k % nbuf] = pltpu.async_copy(
                table.at[srcb.at[pl.ds(k * kchunk, kchunk)]],
                rows[k % nbuf], semg[k % nbuf])

        for j in range(min(nbuf - 1, nchunk)):
            issue_gather(j)

        for k in range(nchunk):
            p = k % nbuf
            nk = k + nbuf - 1
            if nk < nchunk:
                if k >= 1:
                    sct_dsc[nk % nbuf].wait()
                issue_gather(nk)
            gat_dsc[p].wait()

            rp = rows[p]
            koff = k * kchunk

            @plsc.parallel_loop(0, kchunk, unroll=8)
            def _scale(e, _rp=rp, _koff=koff):
                w = plsc.load_gather(
                    ewb, [jnp.full((LANES,), e + _koff, jnp.int32)])
                for cc in range(nsub):
                    sl = pl.ds(cc * LANES, LANES)
                    _rp[e, sl] = _rp[e, sl] * w

            sct_dsc[p] = pltpu.async_copy(rp, acc.at[dstb.at[k]], sems[p],
                                          add=True)

        for k in range(max(0, nchunk - nbuf), nchunk):
            sct_dsc[k % nbuf].wait()
        plsc.subcore_barrier()

        if extract_cols:
            pltpu.sync_copy(acc.at[pl.ds(s * RPS, RPS)],
                            rows[0].at[pl.ds(0, RPS)])
            lane = lax.iota(jnp.int32, LANES)

            @plsc.parallel_loop(0, RPS // LANES, unroll=4)
            def _extract(g):
                r = g * LANES + lane
                for j in range(2):
                    v = plsc.load_gather(
                        rows[0], [r, jnp.full((LANES,), j, jnp.int32)])
                    lbuf[j, pl.ds(g * LANES, LANES)] = v

            for j in range(2):
                pltpu.sync_copy(lbuf.at[j],
                                out.at[c * 2 + j, pl.ds(s * RPS, RPS)])
        else:
            pltpu.sync_copy(acc.at[pl.ds(s * RPS, RPS)],
                            out.at[pl.ds(c * NP + s * RPS, RPS)])

    return body


_l1_seg = _seg_kernel(H, K1, NCH1, 2, False)
_l2_seg = _seg_kernel(W2P, K2, NCH2, 3, True)


def _mm_body(x_ref, w_ref, o_ref):
    o_ref[...] = jnp.dot(x_ref[...], w_ref[...],
                         preferred_element_type=jnp.float32)


_mm = pl.pallas_call(
    _mm_body,
    out_shape=jax.ShapeDtypeStruct((N, H), jnp.float32),
)


def _mid_body(p_ref, w2_ref, o_ref):
    agg = p_ref[0:N, :] + p_ref[NP:NP + N, :]
    nrm = jnp.sqrt(jnp.sum(agg * agg, axis=1, keepdims=True))
    agg = agg / jnp.maximum(nrm, 1e-12)
    h1 = jnp.maximum(agg, 0.0)
    o_ref[...] = jnp.dot(h1, w2_ref[...], preferred_element_type=jnp.float32)


_mid = pl.pallas_call(
    _mid_body,
    out_shape=jax.ShapeDtypeStruct((N, W2P), jnp.float32),
)


def _loss_body(lp_ref, lab_ref, mask_ref, w1_ref, loss_ref, acc_ref):
    l0 = lp_ref[0:1, 0:N] + lp_ref[2:3, 0:N]
    l1 = lp_ref[1:2, 0:N] + lp_ref[3:4, 0:N]
    mx = jnp.maximum(l0, l1)
    lse = mx + jnp.log(jnp.exp(l0 - mx) + jnp.exp(l1 - mx))
    lab1 = jnp.reshape(lab_ref[...], (1, N)) == 1
    sel = jnp.where(lab1, l1, l0)
    ce = lse - sel
    mk = jnp.reshape(mask_ref[...], (1, N))
    mm = mk / (jnp.sum(mk) / N)
    loss = WD * 0.5 * jnp.sum(w1_ref[...] * w1_ref[...])
    loss_ref[0] = loss + jnp.sum(ce * mm) / N
    correct = ((l1 > l0) == lab1).astype(jnp.float32)
    acc_ref[0] = jnp.sum(correct * mm) / N


_loss = pl.pallas_call(
    _loss_body,
    out_shape=[jax.ShapeDtypeStruct((1,), jnp.float32),
               jax.ShapeDtypeStruct((1,), jnp.float32)],
    out_specs=[pl.BlockSpec(memory_space=pltpu.SMEM),
               pl.BlockSpec(memory_space=pltpu.SMEM)],
)


def kernel(x, edge_index, edge_weight, labels, mask, W1, W2):
    zeros = jnp.zeros((ZR, H), jnp.float32)
    pre1 = _mm(x, W1)
    part1 = _l1_seg(pre1, edge_index, edge_weight, zeros)
    w2p = jnp.zeros((H, W2P), jnp.float32).at[:, :C].set(W2)
    pre2 = _mid(part1, w2p)
    part2 = _l2_seg(pre2, edge_index, edge_weight, zeros[:, :W2P])
    loss, acc = _loss(part2, labels, mask, W1)
    return (loss[0], acc[0])


# speedup vs baseline: 1.2230x; 1.2230x over previous
import functools

import jax
import jax.numpy as jnp
from jax import lax
from jax.experimental import pallas as pl
from jax.experimental.pallas import tpu as pltpu
from jax.experimental.pallas import tpu_sc as plsc

N = 10000
E = 320000
D = 128
H = 64
C = 2
W2P = 16
WD = 0.0005

NC = 2
NS = 16
LANES = 16
NW = NC * NS
EPW = E // NW
NP = 10240
RPS = NP // NS
ZR = 80

K1 = 400
NCH1 = EPW // K1
K2 = 2000
NCH2 = EPW // K2

_mesh = plsc.VectorSubcoreMesh(core_axis_name="c", subcore_axis_name="s")


def _seg_kernel(width, kchunk, nchunk, nbuf, extract_cols):
    nsub = width // LANES
    if extract_cols:
        out_type = jax.ShapeDtypeStruct((2 * NC, NP), jnp.float32)
    else:
        out_type = jax.ShapeDtypeStruct((NC * NP, width), jnp.float32)
    scratch = [
        pltpu.VMEM_SHARED((NP, width), jnp.float32),
        pltpu.VMEM((ZR, width), jnp.float32),
        pltpu.VMEM((2, RPS), jnp.float32),
    ]
    scratch += [pltpu.VMEM((kchunk, width), jnp.float32)
                for _ in range(nbuf)]
    scratch += [pltpu.VMEM((2, kchunk), jnp.int32) for _ in range(4)]
    scratch += [pltpu.VMEM((kchunk,), jnp.float32) for _ in range(4)]
    scratch += [pltpu.SemaphoreType.DMA] * (5 + 2 * nbuf)

    @functools.partial(
        pl.kernel,
        out_type=out_type,
        mesh=_mesh,
        compiler_params=pltpu.CompilerParams(
            needs_layout_passes=False, use_tc_tiling_on_sc=False),
        scratch_types=scratch,
    )
    def body(table, eih, ewh, out, acc, zbuf, lbuf, *rest):
        rows = list(rest[:nbuf])
        sdv = list(rest[nbuf:nbuf + 4])
        ewv = list(rest[nbuf + 4:nbuf + 8])
        semi = list(rest[nbuf + 8:nbuf + 12])
        mz = rest[nbuf + 12]
        semg = list(rest[nbuf + 13:nbuf + 13 + nbuf])
        sems = list(rest[nbuf + 13 + nbuf:nbuf + 13 + 2 * nbuf])

        c = lax.axis_index("c")
        s = lax.axis_index("s")
        base = (s * NC + c) * EPW

        idx_dsc = [None] * 4
        gat_dsc = [None] * nbuf
        sct_dsc = [None] * nbuf

        def issue_idx(k):
            q = k & 3
            off = base + k * kchunk
            idx_dsc[q] = (
                pltpu.async_copy(eih.at[:, pl.ds(off, kchunk)], sdv[q],
                                 semi[q]),
                pltpu.async_copy(ewh.at[pl.ds(off, kchunk)], ewv[q], semi[q]),
            )

        def wait_idx(k):
            for dsc in idx_dsc[k & 3]:
                dsc.wait()

        def issue_gather(k):
            gat_dsc[k % nbuf] = pltpu.async_copy(
                table.at[sdv[k & 3].at[0]], rows[k % nbuf], semg[k % nbuf])

        for k in range(min(3, nchunk)):
            issue_idx(k)

        zero16 = jnp.zeros((LANES,), jnp.float32)

        @plsc.parallel_loop(0, ZR, unroll=8)
        def _zrow(i):
            for cc in range(nsub):
                zbuf[i, pl.ds(cc * LANES, LANES)] = zero16

        zdsc = [pltpu.async_copy(zbuf, acc.at[pl.ds(s * RPS + j * ZR, ZR)],
                                 mz) for j in range(RPS // ZR)]
        for dsc in zdsc:
            dsc.wait()
        plsc.subcore_barrier()

        for k in range(min(nbuf - 1, nchunk)):
            wait_idx(k)
            issue_gather(k)

        for k in range(nchunk):
            p = k % nbuf
            q = k & 3
            gat_dsc[p].wait()

            rp = rows[p]
            ewq = ewv[q]

            @plsc.parallel_loop(0, kchunk, unroll=8)
            def _scale(e, _rp=rp, _ew=ewq):
                w = plsc.load_gather(_ew, [jnp.full((LANES,), e, jnp.int32)])
                for cc in range(nsub):
                    sl = pl.ds(cc * LANES, LANES)
                    _rp[e, sl] = _rp[e, sl] * w

            sct_dsc[p] = pltpu.async_copy(rp, acc.at[sdv[q].at[1]], sems[p],
                                          add=True)
            if k + nbuf - 1 < nchunk:
                wait_idx(k + nbuf - 1)
                if k >= 1:
                    sct_dsc[(k - 1) % nbuf].wait()
                issue_gather(k + nbuf - 1)
            if k + 3 < nchunk:
                issue_idx(k + 3)

        for k in range(max(0, nchunk - nbuf), nchunk):
            sct_dsc[k % nbuf].wait()
        plsc.subcore_barrier()

        if extract_cols:
            pltpu.sync_copy(acc.at[pl.ds(s * RPS, RPS)],
                            rows[0].at[pl.ds(0, RPS)])
            lane = lax.iota(jnp.int32, LANES)

            @plsc.parallel_loop(0, RPS // LANES, unroll=4)
            def _extract(g):
                r = g * LANES + lane
                for j in range(2):
                    v = plsc.load_gather(
                        rows[0], [r, jnp.full((LANES,), j, jnp.int32)])
                    lbuf[j, pl.ds(g * LANES, LANES)] = v

            for j in range(2):
                pltpu.sync_copy(lbuf.at[j],
                                out.at[c * 2 + j, pl.ds(s * RPS, RPS)])
        else:
            pltpu.sync_copy(acc.at[pl.ds(s * RPS, RPS)],
                            out.at[pl.ds(c * NP + s * RPS, RPS)])

    return body


_l1_seg = _seg_kernel(H, K1, NCH1, 3, False)
_l2_seg = _seg_kernel(W2P, K2, NCH2, 2, True)


def _mm_body(x_ref, w_ref, o_ref):
    o_ref[...] = jnp.dot(x_ref[...], w_ref[...],
                         preferred_element_type=jnp.float32)


_mm = pl.pallas_call(
    _mm_body,
    out_shape=jax.ShapeDtypeStruct((N, H), jnp.float32),
)


def _mid_body(p_ref, w2_ref, o_ref):
    agg = p_ref[0:N, :] + p_ref[NP:NP + N, :]
    nrm = jnp.sqrt(jnp.sum(agg * agg, axis=1, keepdims=True))
    agg = agg / jnp.maximum(nrm, 1e-12)
    h1 = jnp.maximum(agg, 0.0)
    o_ref[...] = jnp.dot(h1, w2_ref[...], preferred_element_type=jnp.float32)


_mid = pl.pallas_call(
    _mid_body,
    out_shape=jax.ShapeDtypeStruct((N, W2P), jnp.float32),
)


def _loss_body(lp_ref, lab_ref, mask_ref, w1_ref, loss_ref, acc_ref):
    l0 = lp_ref[0:1, 0:N] + lp_ref[2:3, 0:N]
    l1 = lp_ref[1:2, 0:N] + lp_ref[3:4, 0:N]
    mx = jnp.maximum(l0, l1)
    lse = mx + jnp.log(jnp.exp(l0 - mx) + jnp.exp(l1 - mx))
    lab1 = jnp.reshape(lab_ref[...], (1, N)) == 1
    sel = jnp.where(lab1, l1, l0)
    ce = lse - sel
    mk = jnp.reshape(mask_ref[...], (1, N))
    mm = mk / (jnp.sum(mk) / N)
    loss = WD * 0.5 * jnp.sum(w1_ref[...] * w1_ref[...])
    loss_ref[0] = loss + jnp.sum(ce * mm) / N
    correct = ((l1 > l0) == lab1).astype(jnp.float32)
    acc_ref[0] = jnp.sum(correct * mm) / N


_loss = pl.pallas_call(
    _loss_body,
    out_shape=[jax.ShapeDtypeStruct((1,), jnp.float32),
               jax.ShapeDtypeStruct((1,), jnp.float32)],
    out_specs=[pl.BlockSpec(memory_space=pltpu.SMEM),
               pl.BlockSpec(memory_space=pltpu.SMEM)],
)


def kernel(x, edge_index, edge_weight, labels, mask, W1, W2):
    pre1 = _mm(x, W1)
    part1 = _l1_seg(pre1, edge_index, edge_weight)
    w2p = jnp.zeros((H, W2P), jnp.float32).at[:, :C].set(W2)
    pre2 = _mid(part1, w2p)
    part2 = _l2_seg(pre2, edge_index, edge_weight)
    loss, acc = _loss(part2, labels, mask, W1)
    return (loss[0], acc[0])


# speedup vs baseline: 1.2988x; 1.0620x over previous
import functools

import jax
import jax.numpy as jnp
from jax import lax
from jax.experimental import pallas as pl
from jax.experimental.pallas import tpu as pltpu
from jax.experimental.pallas import tpu_sc as plsc

N = 10000
E = 320000
D = 128
H = 64
C = 2
W2P = 16
WD = 0.0005

NC = 2
NS = 16
LANES = 16
NW = NC * NS
EPW = E // NW
NP = 10240
RPS = NP // NS
ZR = 80

K1 = 400
NCH1 = EPW // K1
K2 = 1000
NCH2 = EPW // K2

_mesh = plsc.VectorSubcoreMesh(core_axis_name="c", subcore_axis_name="s")


def _seg_kernel(width, kchunk, nchunk, nbuf, extract_cols):
    nsub = width // LANES
    if extract_cols:
        out_type = jax.ShapeDtypeStruct((2 * NC, NP), jnp.float32)
    else:
        out_type = jax.ShapeDtypeStruct((NC * NP, width), jnp.float32)
    scratch = [
        pltpu.VMEM_SHARED((NP, width), jnp.float32),
        pltpu.VMEM((ZR, width), jnp.float32),
        pltpu.VMEM((2, RPS), jnp.float32),
    ]
    scratch += [pltpu.VMEM((kchunk, width), jnp.float32)
                for _ in range(nbuf)]
    scratch += [pltpu.VMEM((2, kchunk), jnp.int32) for _ in range(4)]
    scratch += [pltpu.VMEM((kchunk,), jnp.float32) for _ in range(4)]
    scratch += [pltpu.SemaphoreType.DMA] * (5 + 2 * nbuf)

    @functools.partial(
        pl.kernel,
        out_type=out_type,
        mesh=_mesh,
        compiler_params=pltpu.CompilerParams(
            needs_layout_passes=False, use_tc_tiling_on_sc=False),
        scratch_types=scratch,
    )
    def body(table, eih, ewh, out, acc, zbuf, lbuf, *rest):
        rows = list(rest[:nbuf])
        sdv = list(rest[nbuf:nbuf + 4])
        ewv = list(rest[nbuf + 4:nbuf + 8])
        semi = list(rest[nbuf + 8:nbuf + 12])
        mz = rest[nbuf + 12]
        semg = list(rest[nbuf + 13:nbuf + 13 + nbuf])
        sems = list(rest[nbuf + 13 + nbuf:nbuf + 13 + 2 * nbuf])

        c = lax.axis_index("c")
        s = lax.axis_index("s")
        base = (s * NC + c) * EPW

        idx_dsc = [None] * 4
        gat_dsc = [None] * nbuf
        sct_dsc = [None] * nbuf

        def issue_idx(k):
            q = k & 3
            off = base + k * kchunk
            idx_dsc[q] = (
                pltpu.async_copy(eih.at[:, pl.ds(off, kchunk)], sdv[q],
                                 semi[q]),
                pltpu.async_copy(ewh.at[pl.ds(off, kchunk)], ewv[q], semi[q]),
            )

        def wait_idx(k):
            for dsc in idx_dsc[k & 3]:
                dsc.wait()

        def issue_gather(k):
            gat_dsc[k % nbuf] = pltpu.async_copy(
                table.at[sdv[k & 3].at[0]], rows[k % nbuf], semg[k % nbuf])

        for k in range(min(3, nchunk)):
            issue_idx(k)

        zero16 = jnp.zeros((LANES,), jnp.float32)

        @plsc.parallel_loop(0, ZR, unroll=8)
        def _zrow(i):
            for cc in range(nsub):
                zbuf[i, pl.ds(cc * LANES, LANES)] = zero16

        zdsc = [pltpu.async_copy(zbuf, acc.at[pl.ds(s * RPS + j * ZR, ZR)],
                                 mz) for j in range(RPS // ZR)]
        for dsc in zdsc:
            dsc.wait()
        plsc.subcore_barrier()

        for k in range(min(nbuf - 1, nchunk)):
            wait_idx(k)
            issue_gather(k)

        for k in range(nchunk):
            p = k % nbuf
            q = k & 3
            gat_dsc[p].wait()

            rp = rows[p]
            ewq = ewv[q]

            @plsc.parallel_loop(0, kchunk, unroll=8)
            def _scale(e, _rp=rp, _ew=ewq):
                w = plsc.load_gather(_ew, [jnp.full((LANES,), e, jnp.int32)])
                for cc in range(nsub):
                    sl = pl.ds(cc * LANES, LANES)
                    _rp[e, sl] = _rp[e, sl] * w

            sct_dsc[p] = pltpu.async_copy(rp, acc.at[sdv[q].at[1]], sems[p],
                                          add=True)
            if k + nbuf - 1 < nchunk:
                wait_idx(k + nbuf - 1)
                if k >= 1:
                    sct_dsc[(k - 1) % nbuf].wait()
                issue_gather(k + nbuf - 1)
            if k + 3 < nchunk:
                issue_idx(k + 3)

        for k in range(max(0, nchunk - nbuf), nchunk):
            sct_dsc[k % nbuf].wait()
        plsc.subcore_barrier()

        if extract_cols:
            pltpu.sync_copy(acc.at[pl.ds(s * RPS, RPS)],
                            rows[0].at[pl.ds(0, RPS)])
            lane = lax.iota(jnp.int32, LANES)

            @plsc.parallel_loop(0, RPS // LANES, unroll=4)
            def _extract(g):
                r = g * LANES + lane
                for j in range(2):
                    v = plsc.load_gather(
                        rows[0], [r, jnp.full((LANES,), j, jnp.int32)])
                    lbuf[j, pl.ds(g * LANES, LANES)] = v

            for j in range(2):
                pltpu.sync_copy(lbuf.at[j],
                                out.at[c * 2 + j, pl.ds(s * RPS, RPS)])
        else:
            pltpu.sync_copy(acc.at[pl.ds(s * RPS, RPS)],
                            out.at[pl.ds(c * NP + s * RPS, RPS)])

    return body


_l1_seg = _seg_kernel(H, K1, NCH1, 3, False)
_l2_seg = _seg_kernel(W2P, K2, NCH2, 3, True)


def _mm_body(x_ref, w_ref, o_ref):
    o_ref[...] = jnp.dot(x_ref[...], w_ref[...],
                         preferred_element_type=jnp.float32)


_mm = pl.pallas_call(
    _mm_body,
    out_shape=jax.ShapeDtypeStruct((N, H), jnp.float32),
)


def _mid_body(p_ref, w2_ref, o_ref):
    agg = p_ref[0:N, :] + p_ref[NP:NP + N, :]
    nrm = jnp.sqrt(jnp.sum(agg * agg, axis=1, keepdims=True))
    agg = agg / jnp.maximum(nrm, 1e-12)
    h1 = jnp.maximum(agg, 0.0)
    o_ref[...] = jnp.dot(h1, w2_ref[...], preferred_element_type=jnp.float32)


_mid = pl.pallas_call(
    _mid_body,
    out_shape=jax.ShapeDtypeStruct((N, W2P), jnp.float32),
)


def _loss_body(lp_ref, lab_ref, mask_ref, w1_ref, loss_ref, acc_ref):
    l0 = lp_ref[0:1, 0:N] + lp_ref[2:3, 0:N]
    l1 = lp_ref[1:2, 0:N] + lp_ref[3:4, 0:N]
    mx = jnp.maximum(l0, l1)
    lse = mx + jnp.log(jnp.exp(l0 - mx) + jnp.exp(l1 - mx))
    lab1 = jnp.reshape(lab_ref[...], (1, N)) == 1
    sel = jnp.where(lab1, l1, l0)
    ce = lse - sel
    mk = jnp.reshape(mask_ref[...], (1, N))
    mm = mk / (jnp.sum(mk) / N)
    loss = WD * 0.5 * jnp.sum(w1_ref[...] * w1_ref[...])
    loss_ref[0] = loss + jnp.sum(ce * mm) / N
    correct = ((l1 > l0) == lab1).astype(jnp.float32)
    acc_ref[0] = jnp.sum(correct * mm) / N


_loss = pl.pallas_call(
    _loss_body,
    out_shape=[jax.ShapeDtypeStruct((1,), jnp.float32),
               jax.ShapeDtypeStruct((1,), jnp.float32)],
    out_specs=[pl.BlockSpec(memory_space=pltpu.SMEM),
               pl.BlockSpec(memory_space=pltpu.SMEM)],
)


def kernel(x, edge_index, edge_weight, labels, mask, W1, W2):
    pre1 = _mm(x, W1)
    part1 = _l1_seg(pre1, edge_index, edge_weight)
    w2p = jnp.zeros((H, W2P), jnp.float32).at[:, :C].set(W2)
    pre2 = _mid(part1, w2p)
    part2 = _l2_seg(pre2, edge_index, edge_weight)
    loss, acc = _loss(part2, labels, mask, W1)
    return (loss[0], acc[0])
